# Initial kernel scaffold; baseline (speedup 1.0000x reference)
import jax
import jax.numpy as jnp
from jax.experimental import pallas as pl


def kernel(x, edge_index, batch, W1, b1, W2, b2, Wl, bl):
    raise NotImplementedError("write your pallas kernel here")


# speedup vs baseline: 2.4099x; 2.4099x over previous
import functools

import jax
import jax.numpy as jnp
from jax import lax
from jax.experimental import pallas as pl
from jax.experimental.pallas import tpu as pltpu
from jax.experimental.pallas import tpu_sc as plsc

NNODE = 10000
NGRAPH = 16
NDIM = 128
NTOT = NNODE + NGRAPH
RP = 10240
ZR = RP // 16

NWORK = 32
CHUNK = 128


def _make_sc_agg(ch_per_w):
    mesh = plsc.VectorSubcoreMesh(core_axis_name="c", subcore_axis_name="s")

    @functools.partial(
        pl.kernel,
        out_type=jax.ShapeDtypeStruct((2, RP, NDIM), jnp.float32),
        mesh=mesh,
        scratch_types=[
            pltpu.VMEM((ch_per_w, CHUNK), jnp.int32),
            pltpu.VMEM((ch_per_w, CHUNK), jnp.int32),
            pltpu.VMEM((CHUNK, NDIM), jnp.float32),
            pltpu.VMEM_SHARED((RP, NDIM), jnp.float32),
            pltpu.SemaphoreType.DMA,
        ],
    )
    def sc_agg(h_hbm, src_hbm, dst_hbm, zeros_hbm, out_hbm,
               src_v, dst_v, rows_v, agg_sh, sem):
        cid = lax.axis_index("c")
        sid = lax.axis_index("s")
        w = cid * 16 + sid
        pltpu.sync_copy(zeros_hbm, agg_sh.at[pl.ds(sid * ZR, ZR)])
        pltpu.sync_copy(src_hbm.at[w], src_v)
        pltpu.sync_copy(dst_hbm.at[w], dst_v)
        plsc.subcore_barrier()

        def body(g, carry):
            pltpu.async_copy(h_hbm.at[src_v.at[g]], rows_v, sem).wait()
            pltpu.sync_copy(rows_v, agg_sh.at[dst_v.at[g]], add=True)
            return carry

        lax.fori_loop(0, ch_per_w, body, 0)
        plsc.subcore_barrier()
        pltpu.sync_copy(agg_sh.at[pl.ds(sid * ZR, ZR)],
                        out_hbm.at[cid, pl.ds(sid * ZR, ZR)])

    return sc_agg


def _mlp_body(h_ref, a_ref, w1_ref, b1_ref, w2_ref, b2_ref, o_ref, *, last):
    z = h_ref[...] + a_ref[0] + a_ref[1]
    t = jnp.dot(z, w1_ref[...], preferred_element_type=jnp.float32) + b1_ref[...]
    t = jnp.maximum(t, 0.0)
    y = jnp.dot(t, w2_ref[...], preferred_element_type=jnp.float32) + b2_ref[...]
    if not last:
        y = jnp.maximum(y, 0.0)
    o_ref[...] = y


def _mlp(h, agg, w1t, b1, w2t, b2, last):
    br = 1024
    grid = (RP // br,)
    return pl.pallas_call(
        functools.partial(_mlp_body, last=last),
        grid=grid,
        in_specs=[
            pl.BlockSpec((br, NDIM), lambda i: (i, 0)),
            pl.BlockSpec((2, br, NDIM), lambda i: (0, i, 0)),
            pl.BlockSpec((NDIM, NDIM), lambda i: (0, 0)),
            pl.BlockSpec((1, NDIM), lambda i: (0, 0)),
            pl.BlockSpec((NDIM, NDIM), lambda i: (0, 0)),
            pl.BlockSpec((1, NDIM), lambda i: (0, 0)),
        ],
        out_specs=pl.BlockSpec((br, NDIM), lambda i: (i, 0)),
        out_shape=jax.ShapeDtypeStruct((RP, NDIM), jnp.float32),
    )(h, agg, w1t, b1, w2t, b2)


def _final_body(hv_ref, wl_ref, bl_ref, o_ref):
    z = jnp.maximum(hv_ref[...], 0.0)
    o_ref[...] = (jnp.dot(z, wl_ref[...], preferred_element_type=jnp.float32)
                  + bl_ref[...])


def _final(hv, wlt, bl):
    return pl.pallas_call(
        _final_body,
        out_shape=jax.ShapeDtypeStruct((NGRAPH, NDIM), jnp.float32),
    )(hv, wlt, bl)


def kernel(x, edge_index, batch, W1, b1, W2, b2, Wl, bl):
    n, d = x.shape
    e = edge_index.shape[1]
    idt = jnp.int32
    ar = jnp.arange(n, dtype=idt)
    vb = n + batch.astype(idt)
    src = jnp.concatenate([edge_index[0].astype(idt), ar, vb])
    dst = jnp.concatenate([edge_index[1].astype(idt), vb, ar])
    tot = e + 2 * n
    ch_per_w = -(-tot // (NWORK * CHUNK))
    pad = NWORK * CHUNK * ch_per_w - tot
    src = jnp.concatenate([src, jnp.zeros((pad,), idt)])
    dst = jnp.concatenate([dst, jnp.full((pad,), NTOT, idt)])
    src3 = src.reshape(NWORK, ch_per_w, CHUNK)
    dst3 = dst.reshape(NWORK, ch_per_w, CHUNK)

    h = jnp.zeros((RP, NDIM), jnp.float32).at[:n].set(x)
    zeros = jnp.zeros((ZR, NDIM), jnp.float32)
    w1t = jnp.swapaxes(W1, 1, 2)
    w2t = jnp.swapaxes(W2, 1, 2)
    b1r = b1.reshape(W1.shape[0], 1, NDIM)
    b2r = b2.reshape(W1.shape[0], 1, NDIM)

    sc_agg = _make_sc_agg(ch_per_w)
    nl = W1.shape[0]
    for l in range(nl):
        agg = sc_agg(h, src3, dst3, zeros)
        h = _mlp(h, agg, w1t[l], b1r[l], w2t[l], b2r[l], last=(l == nl - 1))

    hv = lax.slice(h, (n, 0), (n + NGRAPH, NDIM))
    return _final(hv, Wl.T, bl.reshape(1, NDIM))


# speedup vs baseline: 2.5750x; 1.0685x over previous
import functools

import jax
import jax.numpy as jnp
from jax import lax
from jax.experimental import pallas as pl
from jax.experimental.pallas import tpu as pltpu
from jax.experimental.pallas import tpu_sc as plsc

NNODE = 10000
NGRAPH = 16
NDIM = 128
NTOT = NNODE + NGRAPH
RP = 10240
ZR = RP // 16

NWORK = 32
CHUNK = 128


def _make_sc_agg(ch_per_w):
    mesh = plsc.VectorSubcoreMesh(core_axis_name="c", subcore_axis_name="s")

    @functools.partial(
        pl.kernel,
        out_type=jax.ShapeDtypeStruct((2, RP, NDIM), jnp.float32),
        mesh=mesh,
        scratch_types=[
            pltpu.VMEM((2, CHUNK), jnp.int32),
            pltpu.VMEM((2, CHUNK), jnp.int32),
            pltpu.VMEM((CHUNK, NDIM), jnp.float32),
            pltpu.VMEM((CHUNK, NDIM), jnp.float32),
            pltpu.VMEM_SHARED((RP, NDIM), jnp.float32),
            pltpu.SemaphoreType.DMA,
            pltpu.SemaphoreType.DMA,
            pltpu.SemaphoreType.DMA,
            pltpu.SemaphoreType.DMA,
        ],
    )
    def sc_agg(h_hbm, sd_hbm, zeros_hbm, out_hbm,
               idx0_v, idx1_v, rows0_v, rows1_v, agg_sh,
               gsem0, gsem1, isem0, isem1):
        cid = lax.axis_index("c")
        sid = lax.axis_index("s")
        w = cid * 16 + sid
        pltpu.sync_copy(zeros_hbm, agg_sh.at[pl.ds(sid * ZR, ZR)])
        plsc.subcore_barrier()

        idx = (idx0_v, idx1_v)
        rows = (rows0_v, rows1_v)
        gsem = (gsem0, gsem1)
        isem = (isem0, isem1)

        pltpu.sync_copy(sd_hbm.at[w, 0], idx[0])
        pltpu.async_copy(sd_hbm.at[w, 1], idx[1], isem[1])
        pltpu.async_copy(h_hbm.at[idx[0].at[0]], rows[0], gsem[0])

        def body(p, carry):
            for b in (0, 1):
                g = 2 * p + b
                nb = 1 - b
                pltpu.make_async_copy(h_hbm.at[idx[b].at[0]], rows[b],
                                      gsem[b]).wait()

                @pl.when(g + 1 < ch_per_w)
                def _():
                    pltpu.make_async_copy(sd_hbm.at[w, g + 1], idx[nb],
                                          isem[nb]).wait()
                    pltpu.async_copy(h_hbm.at[idx[nb].at[0]], rows[nb],
                                     gsem[nb])

                pltpu.sync_copy(rows[b], agg_sh.at[idx[b].at[1]], add=True)

                @pl.when(g + 2 < ch_per_w)
                def _():
                    pltpu.async_copy(sd_hbm.at[w, g + 2], idx[b], isem[b])
            return carry

        lax.fori_loop(0, ch_per_w // 2, body, 0)
        plsc.subcore_barrier()
        pltpu.sync_copy(agg_sh.at[pl.ds(sid * ZR, ZR)],
                        out_hbm.at[cid, pl.ds(sid * ZR, ZR)])

    return sc_agg


def _mlp_body(h_ref, a_ref, w1_ref, b1_ref, w2_ref, b2_ref, o_ref, *, last):
    z = h_ref[...] + a_ref[0] + a_ref[1]
    t = jnp.dot(z, w1_ref[...], preferred_element_type=jnp.float32) + b1_ref[...]
    t = jnp.maximum(t, 0.0)
    y = jnp.dot(t, w2_ref[...], preferred_element_type=jnp.float32) + b2_ref[...]
    if not last:
        y = jnp.maximum(y, 0.0)
    o_ref[...] = y


def _mlp(h, agg, w1t, b1, w2t, b2, last):
    br = 1024
    grid = (RP // br,)
    return pl.pallas_call(
        functools.partial(_mlp_body, last=last),
        grid=grid,
        in_specs=[
            pl.BlockSpec((br, NDIM), lambda i: (i, 0)),
            pl.BlockSpec((2, br, NDIM), lambda i: (0, i, 0)),
            pl.BlockSpec((NDIM, NDIM), lambda i: (0, 0)),
            pl.BlockSpec((1, NDIM), lambda i: (0, 0)),
            pl.BlockSpec((NDIM, NDIM), lambda i: (0, 0)),
            pl.BlockSpec((1, NDIM), lambda i: (0, 0)),
        ],
        out_specs=pl.BlockSpec((br, NDIM), lambda i: (i, 0)),
        out_shape=jax.ShapeDtypeStruct((RP, NDIM), jnp.float32),
    )(h, agg, w1t, b1, w2t, b2)


def _final_body(hv_ref, wl_ref, bl_ref, o_ref):
    z = jnp.maximum(hv_ref[...], 0.0)
    o_ref[...] = (jnp.dot(z, wl_ref[...], preferred_element_type=jnp.float32)
                  + bl_ref[...])


def _final(hv, wlt, bl):
    return pl.pallas_call(
        _final_body,
        out_shape=jax.ShapeDtypeStruct((NGRAPH, NDIM), jnp.float32),
    )(hv, wlt, bl)


def kernel(x, edge_index, batch, W1, b1, W2, b2, Wl, bl):
    n, d = x.shape
    e = edge_index.shape[1]
    idt = jnp.int32
    ar = jnp.arange(n, dtype=idt)
    vb = n + batch.astype(idt)
    src = jnp.concatenate([edge_index[0].astype(idt), ar, vb])
    dst = jnp.concatenate([edge_index[1].astype(idt), vb, ar])
    tot = e + 2 * n
    ch_per_w = -(-tot // (NWORK * CHUNK))
    ch_per_w += ch_per_w % 2
    pad = NWORK * CHUNK * ch_per_w - tot
    src = jnp.concatenate([src, jnp.zeros((pad,), idt)])
    dst = jnp.concatenate([dst, jnp.full((pad,), NTOT, idt)])
    sd = jnp.stack([src.reshape(NWORK, ch_per_w, CHUNK),
                    dst.reshape(NWORK, ch_per_w, CHUNK)], axis=2)

    h = jnp.zeros((RP, NDIM), jnp.float32).at[:n].set(x)
    zeros = jnp.zeros((ZR, NDIM), jnp.float32)
    w1t = jnp.swapaxes(W1, 1, 2)
    w2t = jnp.swapaxes(W2, 1, 2)
    b1r = b1.reshape(W1.shape[0], 1, NDIM)
    b2r = b2.reshape(W1.shape[0], 1, NDIM)

    sc_agg = _make_sc_agg(ch_per_w)
    nl = W1.shape[0]
    for l in range(nl):
        agg = sc_agg(h, sd, zeros)
        h = _mlp(h, agg, w1t[l], b1r[l], w2t[l], b2r[l], last=(l == nl - 1))

    hv = lax.slice(h, (n, 0), (n + NGRAPH, NDIM))
    return _final(hv, Wl.T, bl.reshape(1, NDIM))


# speedup vs baseline: 3.1086x; 1.2073x over previous
import functools

import jax
import jax.numpy as jnp
from jax import lax
from jax.experimental import pallas as pl
from jax.experimental.pallas import tpu as pltpu
from jax.experimental.pallas import tpu_sc as plsc

NNODE = 10000
NGRAPH = 16
NDIM = 128
NTOT = NNODE + NGRAPH
RP = 10240
ZR = RP // 16

NWORK = 32
CHUNK = 128
BR = 1024


def _make_sc_agg(ch_per_w):
    mesh = plsc.VectorSubcoreMesh(core_axis_name="c", subcore_axis_name="s")

    @functools.partial(
        pl.kernel,
        out_type=jax.ShapeDtypeStruct((2, RP, NDIM), jnp.float32),
        mesh=mesh,
        scratch_types=[
            pltpu.VMEM((2, CHUNK), jnp.int32),
            pltpu.VMEM((2, CHUNK), jnp.int32),
            pltpu.VMEM((CHUNK, NDIM), jnp.float32),
            pltpu.VMEM((CHUNK, NDIM), jnp.float32),
            pltpu.VMEM_SHARED((RP, NDIM), jnp.float32),
            pltpu.SemaphoreType.DMA,
            pltpu.SemaphoreType.DMA,
            pltpu.SemaphoreType.DMA,
            pltpu.SemaphoreType.DMA,
        ],
    )
    def sc_agg(h_hbm, sd_hbm, zeros_hbm, out_hbm,
               idx0_v, idx1_v, rows0_v, rows1_v, agg_sh,
               gsem0, gsem1, isem0, isem1):
        cid = lax.axis_index("c")
        sid = lax.axis_index("s")
        w = cid * 16 + sid
        pltpu.sync_copy(zeros_hbm, agg_sh.at[pl.ds(sid * ZR, ZR)])
        plsc.subcore_barrier()

        idx = (idx0_v, idx1_v)
        rows = (rows0_v, rows1_v)
        gsem = (gsem0, gsem1)
        isem = (isem0, isem1)

        pltpu.sync_copy(sd_hbm.at[w, 0], idx[0])
        pltpu.async_copy(sd_hbm.at[w, 1], idx[1], isem[1])
        pltpu.async_copy(h_hbm.at[idx[0].at[0]], rows[0], gsem[0])

        def body(p, carry):
            for b in (0, 1):
                g = 2 * p + b
                nb = 1 - b
                pltpu.make_async_copy(h_hbm.at[idx[b].at[0]], rows[b],
                                      gsem[b]).wait()

                @pl.when(g + 1 < ch_per_w)
                def _():
                    pltpu.make_async_copy(sd_hbm.at[w, g + 1], idx[nb],
                                          isem[nb]).wait()
                    pltpu.async_copy(h_hbm.at[idx[nb].at[0]], rows[nb],
                                     gsem[nb])

                pltpu.sync_copy(rows[b], agg_sh.at[idx[b].at[1]], add=True)

                @pl.when(g + 2 < ch_per_w)
                def _():
                    pltpu.async_copy(sd_hbm.at[w, g + 2], idx[b], isem[b])
            return carry

        lax.fori_loop(0, ch_per_w // 2, body, 0)
        plsc.subcore_barrier()
        pltpu.sync_copy(agg_sh.at[pl.ds(sid * ZR, ZR)],
                        out_hbm.at[cid, pl.ds(sid * ZR, ZR)])

    return sc_agg


def _mlp_body(h_ref, a_ref, m_ref, p_ref, vc_ref, w1_ref, b1_ref,
              w2_ref, b2_ref, o_ref, vo_ref, *, last):
    i = pl.program_id(0)
    z = h_ref[...] + a_ref[0] + a_ref[1]
    z = z + jnp.dot(m_ref[...], vc_ref[...], preferred_element_type=jnp.float32)
    t = jnp.dot(z, w1_ref[...], preferred_element_type=jnp.float32) + b1_ref[...]
    t = jnp.maximum(t, 0.0)
    y = jnp.dot(t, w2_ref[...], preferred_element_type=jnp.float32) + b2_ref[...]
    if not last:
        y = jnp.maximum(y, 0.0)
    o_ref[...] = y
    part = lax.dot_general(p_ref[...], y, (((0,), (0,)), ((), ())),
                           preferred_element_type=jnp.float32)

    @pl.when(i == 0)
    def _():
        vo_ref[...] = jnp.zeros_like(vo_ref)

    vo_ref[...] += part


def _mlp(h, agg, m, p, vcat, w1t, b1, w2t, b2, last):
    grid = (RP // BR,)
    return pl.pallas_call(
        functools.partial(_mlp_body, last=last),
        grid=grid,
        in_specs=[
            pl.BlockSpec((BR, NDIM), lambda i: (i, 0)),
            pl.BlockSpec((2, BR, NDIM), lambda i: (0, i, 0)),
            pl.BlockSpec((BR, 2 * NGRAPH), lambda i: (i, 0)),
            pl.BlockSpec((BR, 2 * NGRAPH), lambda i: (i, 0)),
            pl.BlockSpec((2 * NGRAPH, NDIM), lambda i: (0, 0)),
            pl.BlockSpec((NDIM, NDIM), lambda i: (0, 0)),
            pl.BlockSpec((1, NDIM), lambda i: (0, 0)),
            pl.BlockSpec((NDIM, NDIM), lambda i: (0, 0)),
            pl.BlockSpec((1, NDIM), lambda i: (0, 0)),
        ],
        out_specs=[
            pl.BlockSpec((BR, NDIM), lambda i: (i, 0)),
            pl.BlockSpec((2 * NGRAPH, NDIM), lambda i: (0, 0)),
        ],
        out_shape=[
            jax.ShapeDtypeStruct((RP, NDIM), jnp.float32),
            jax.ShapeDtypeStruct((2 * NGRAPH, NDIM), jnp.float32),
        ],
    )(h, agg, m, p, vcat, w1t, b1, w2t, b2)


def _vcat0_body(h_ref, p_ref, vo_ref):
    i = pl.program_id(0)

    @pl.when(i == 0)
    def _():
        vo_ref[...] = jnp.zeros_like(vo_ref)

    vo_ref[...] += lax.dot_general(p_ref[...], h_ref[...],
                                   (((0,), (0,)), ((), ())),
                                   preferred_element_type=jnp.float32)


def _vcat0(h, p):
    return pl.pallas_call(
        _vcat0_body,
        grid=(RP // BR,),
        in_specs=[
            pl.BlockSpec((BR, NDIM), lambda i: (i, 0)),
            pl.BlockSpec((BR, 2 * NGRAPH), lambda i: (i, 0)),
        ],
        out_specs=pl.BlockSpec((2 * NGRAPH, NDIM), lambda i: (0, 0)),
        out_shape=jax.ShapeDtypeStruct((2 * NGRAPH, NDIM), jnp.float32),
    )(h, p)


def _final_body(hv_ref, wl_ref, bl_ref, o_ref):
    z = jnp.maximum(hv_ref[...], 0.0)
    o_ref[...] = (jnp.dot(z, wl_ref[...], preferred_element_type=jnp.float32)
                  + bl_ref[...])


def _final(hv, wlt, bl):
    return pl.pallas_call(
        _final_body,
        out_shape=jax.ShapeDtypeStruct((NGRAPH, NDIM), jnp.float32),
    )(hv, wlt, bl)


def kernel(x, edge_index, batch, W1, b1, W2, b2, Wl, bl):
    n, d = x.shape
    e = edge_index.shape[1]
    idt = jnp.int32
    src = edge_index[0].astype(idt)
    dst = edge_index[1].astype(idt)
    ch_per_w = -(-e // (NWORK * CHUNK))
    ch_per_w += ch_per_w % 2
    pad = NWORK * CHUNK * ch_per_w - e
    src = jnp.concatenate([src, jnp.zeros((pad,), idt)])
    dst = jnp.concatenate([dst, jnp.full((pad,), NTOT, idt)])
    sd = jnp.stack([src.reshape(NWORK, ch_per_w, CHUNK),
                    dst.reshape(NWORK, ch_per_w, CHUNK)], axis=2)

    h = jnp.zeros((RP, NDIM), jnp.float32).at[:n].set(x)
    zeros = jnp.zeros((ZR, NDIM), jnp.float32)
    rows_i = jnp.arange(RP, dtype=idt)
    bfull = jnp.where(rows_i < n, batch.astype(idt)[jnp.minimum(rows_i, n - 1)], -1)
    m_real = jax.nn.one_hot(bfull, NGRAPH, dtype=jnp.float32)
    m_virt = jax.nn.one_hot(
        jnp.where((rows_i >= n) & (rows_i < n + NGRAPH), rows_i - n, -1),
        NGRAPH, dtype=jnp.float32)
    m = jnp.concatenate([m_real, m_virt], axis=1)
    p = jnp.concatenate([m_virt, m_real], axis=1)

    w1t = jnp.swapaxes(W1, 1, 2)
    w2t = jnp.swapaxes(W2, 1, 2)
    nl = W1.shape[0]
    b1r = b1.reshape(nl, 1, NDIM)
    b2r = b2.reshape(nl, 1, NDIM)

    sc_agg = _make_sc_agg(ch_per_w)
    vcat = _vcat0(h, p)
    for l in range(nl):
        agg = sc_agg(h, sd, zeros)
        h, vcat = _mlp(h, agg, m, p, vcat, w1t[l], b1r[l], w2t[l], b2r[l],
                       last=(l == nl - 1))

    hv = lax.slice(vcat, (0, 0), (NGRAPH, NDIM))
    return _final(hv, Wl.T, bl.reshape(1, NDIM))


# speedup vs baseline: 10.0004x; 3.2170x over previous
import functools

import jax
import jax.numpy as jnp
from jax import lax
from jax.experimental import pallas as pl
from jax.experimental.pallas import tpu as pltpu
from jax.experimental.pallas import tpu_sc as plsc

NNODE = 10000
NGRAPH = 16
NDIM = 128
NTOT = NNODE + NGRAPH
RP = 10240
ZR = RP // 16

NWORK = 32
CHUNK = 128
BR = 1024


def _make_sc_agg(ch_per_w):
    mesh = plsc.VectorSubcoreMesh(core_axis_name="c", subcore_axis_name="s")

    @functools.partial(
        pl.kernel,
        out_type=jax.ShapeDtypeStruct((2, RP, NDIM), jnp.float32),
        mesh=mesh,
        scratch_types=[
            pltpu.VMEM((2, CHUNK), jnp.int32),
            pltpu.VMEM((2, CHUNK), jnp.int32),
            pltpu.VMEM((CHUNK, NDIM), jnp.float32),
            pltpu.VMEM((CHUNK, NDIM), jnp.float32),
            pltpu.VMEM_SHARED((RP, NDIM), jnp.float32),
            pltpu.SemaphoreType.DMA,
            pltpu.SemaphoreType.DMA,
            pltpu.SemaphoreType.DMA,
            pltpu.SemaphoreType.DMA,
        ],
    )
    def sc_agg(h_hbm, sd_hbm, zeros_hbm, out_hbm,
               idx0_v, idx1_v, rows0_v, rows1_v, agg_sh,
               gsem0, gsem1, isem0, isem1):
        cid = lax.axis_index("c")
        sid = lax.axis_index("s")
        w = cid * 16 + sid
        pltpu.sync_copy(zeros_hbm, agg_sh.at[pl.ds(sid * ZR, ZR)])
        plsc.subcore_barrier()

        idx = (idx0_v, idx1_v)
        rows = (rows0_v, rows1_v)
        gsem = (gsem0, gsem1)
        isem = (isem0, isem1)

        pltpu.sync_copy(sd_hbm.at[w, 0], idx[0])
        pltpu.async_copy(sd_hbm.at[w, 1], idx[1], isem[1])
        pltpu.async_copy(h_hbm.at[idx[0].at[0]], rows[0], gsem[0])

        def body(p, carry):
            for b in (0, 1):
                g = 2 * p + b
                nb = 1 - b
                pltpu.make_async_copy(h_hbm.at[idx[b].at[0]], rows[b],
                                      gsem[b]).wait()

                @pl.when(g + 1 < ch_per_w)
                def _():
                    pltpu.make_async_copy(sd_hbm.at[w, g + 1], idx[nb],
                                          isem[nb]).wait()
                    pltpu.async_copy(h_hbm.at[idx[nb].at[0]], rows[nb],
                                     gsem[nb])

                pltpu.sync_copy(rows[b], agg_sh.at[idx[b].at[1]], add=True)

                @pl.when(g + 2 < ch_per_w)
                def _():
                    pltpu.async_copy(sd_hbm.at[w, g + 2], idx[b], isem[b])
            return carry

        lax.fori_loop(0, ch_per_w // 2, body, 0)
        plsc.subcore_barrier()
        pltpu.sync_copy(agg_sh.at[pl.ds(sid * ZR, ZR)],
                        out_hbm.at[cid, pl.ds(sid * ZR, ZR)])

    return sc_agg


def _mlp_body(h_ref, a_ref, m_ref, p_ref, vc_ref, w1_ref, b1_ref,
              w2_ref, b2_ref, o_ref, vo_ref, *, last):
    i = pl.program_id(0)
    z = h_ref[...] + a_ref[0] + a_ref[1]
    z = z + jnp.dot(m_ref[...], vc_ref[...], preferred_element_type=jnp.float32)
    t = jnp.dot(z, w1_ref[...], preferred_element_type=jnp.float32) + b1_ref[...]
    t = jnp.maximum(t, 0.0)
    y = jnp.dot(t, w2_ref[...], preferred_element_type=jnp.float32) + b2_ref[...]
    if not last:
        y = jnp.maximum(y, 0.0)
    o_ref[...] = y
    part = lax.dot_general(p_ref[...], y, (((0,), (0,)), ((), ())),
                           preferred_element_type=jnp.float32)

    @pl.when(i == 0)
    def _():
        vo_ref[...] = jnp.zeros_like(vo_ref)

    vo_ref[...] += part


def _mlp(h, agg, m, p, vcat, w1t, b1, w2t, b2, last):
    grid = (RP // BR,)
    return pl.pallas_call(
        functools.partial(_mlp_body, last=last),
        grid=grid,
        in_specs=[
            pl.BlockSpec((BR, NDIM), lambda i: (i, 0)),
            pl.BlockSpec((2, BR, NDIM), lambda i: (0, i, 0)),
            pl.BlockSpec((BR, 2 * NGRAPH), lambda i: (i, 0)),
            pl.BlockSpec((BR, 2 * NGRAPH), lambda i: (i, 0)),
            pl.BlockSpec((2 * NGRAPH, NDIM), lambda i: (0, 0)),
            pl.BlockSpec((NDIM, NDIM), lambda i: (0, 0)),
            pl.BlockSpec((1, NDIM), lambda i: (0, 0)),
            pl.BlockSpec((NDIM, NDIM), lambda i: (0, 0)),
            pl.BlockSpec((1, NDIM), lambda i: (0, 0)),
        ],
        out_specs=[
            pl.BlockSpec((BR, NDIM), lambda i: (i, 0)),
            pl.BlockSpec((2 * NGRAPH, NDIM), lambda i: (0, 0)),
        ],
        out_shape=[
            jax.ShapeDtypeStruct((RP, NDIM), jnp.float32),
            jax.ShapeDtypeStruct((2 * NGRAPH, NDIM), jnp.float32),
        ],
    )(h, agg, m, p, vcat, w1t, b1, w2t, b2)


def _vcat0_body(h_ref, p_ref, vo_ref):
    i = pl.program_id(0)

    @pl.when(i == 0)
    def _():
        vo_ref[...] = jnp.zeros_like(vo_ref)

    vo_ref[...] += lax.dot_general(p_ref[...], h_ref[...],
                                   (((0,), (0,)), ((), ())),
                                   preferred_element_type=jnp.float32)


def _vcat0(h, p):
    return pl.pallas_call(
        _vcat0_body,
        grid=(RP // BR,),
        in_specs=[
            pl.BlockSpec((BR, NDIM), lambda i: (i, 0)),
            pl.BlockSpec((BR, 2 * NGRAPH), lambda i: (i, 0)),
        ],
        out_specs=pl.BlockSpec((2 * NGRAPH, NDIM), lambda i: (0, 0)),
        out_shape=jax.ShapeDtypeStruct((2 * NGRAPH, NDIM), jnp.float32),
    )(h, p)


def _final_body(hv_ref, wl_ref, bl_ref, o_ref):
    z = jnp.maximum(hv_ref[...], 0.0)
    o_ref[...] = (jnp.dot(z, wl_ref[...], preferred_element_type=jnp.float32)
                  + bl_ref[...])


def _final(hv, wlt, bl):
    return pl.pallas_call(
        _final_body,
        out_shape=jax.ShapeDtypeStruct((NGRAPH, NDIM), jnp.float32),
    )(hv, wlt, bl)


def kernel(x, edge_index, batch, W1, b1, W2, b2, Wl, bl):
    n, d = x.shape
    e = edge_index.shape[1]
    idt = jnp.int32
    src = edge_index[0].astype(idt)
    dst = edge_index[1].astype(idt)
    ch_per_w = -(-e // (NWORK * CHUNK))
    ch_per_w += ch_per_w % 2
    pad = NWORK * CHUNK * ch_per_w - e
    pk = jnp.arange(pad, dtype=idt)
    src = jnp.concatenate([src, (pk * 131) % jnp.int32(n)])
    dst = jnp.concatenate([dst, NTOT + pk % jnp.int32(RP - NTOT)])
    sd = jnp.stack([src.reshape(NWORK, ch_per_w, CHUNK),
                    dst.reshape(NWORK, ch_per_w, CHUNK)], axis=2)

    h = jnp.zeros((RP, NDIM), jnp.float32).at[:n].set(x)
    zeros = jnp.zeros((ZR, NDIM), jnp.float32)
    rows_i = jnp.arange(RP, dtype=idt)
    bfull = jnp.where(rows_i < n, batch.astype(idt)[jnp.minimum(rows_i, n - 1)], -1)
    m_real = jax.nn.one_hot(bfull, NGRAPH, dtype=jnp.float32)
    m_virt = jax.nn.one_hot(
        jnp.where((rows_i >= n) & (rows_i < n + NGRAPH), rows_i - n, -1),
        NGRAPH, dtype=jnp.float32)
    m = jnp.concatenate([m_real, m_virt], axis=1)
    p = jnp.concatenate([m_virt, m_real], axis=1)

    w1t = jnp.swapaxes(W1, 1, 2)
    w2t = jnp.swapaxes(W2, 1, 2)
    nl = W1.shape[0]
    b1r = b1.reshape(nl, 1, NDIM)
    b2r = b2.reshape(nl, 1, NDIM)

    sc_agg = _make_sc_agg(ch_per_w)
    vcat = _vcat0(h, p)
    for l in range(nl):
        agg = sc_agg(h, sd, zeros)
        h, vcat = _mlp(h, agg, m, p, vcat, w1t[l], b1r[l], w2t[l], b2r[l],
                       last=(l == nl - 1))

    hv = lax.slice(vcat, (0, 0), (NGRAPH, NDIM))
    return _final(hv, Wl.T, bl.reshape(1, NDIM))


# speedup vs baseline: 10.1133x; 1.0113x over previous
import functools

import jax
import jax.numpy as jnp
from jax import lax
from jax.experimental import pallas as pl
from jax.experimental.pallas import tpu as pltpu
from jax.experimental.pallas import tpu_sc as plsc

NNODE = 10000
NGRAPH = 16
NDIM = 128
NTOT = NNODE + NGRAPH
RP = 10240
ZR = RP // 16

NWORK = 32
CHUNK = 128
BR = 1024


def _make_sc_agg(ch_per_w):
    mesh = plsc.VectorSubcoreMesh(core_axis_name="c", subcore_axis_name="s")

    @functools.partial(
        pl.kernel,
        out_type=jax.ShapeDtypeStruct((2, RP, NDIM), jnp.float32),
        mesh=mesh,
        scratch_types=[
            pltpu.VMEM((2, CHUNK), jnp.int32),
            pltpu.VMEM((2, CHUNK), jnp.int32),
            pltpu.VMEM((CHUNK, NDIM), jnp.float32),
            pltpu.VMEM((CHUNK, NDIM), jnp.float32),
            pltpu.VMEM_SHARED((RP, NDIM), jnp.float32),
            pltpu.SemaphoreType.DMA,
            pltpu.SemaphoreType.DMA,
            pltpu.SemaphoreType.DMA,
            pltpu.SemaphoreType.DMA,
        ],
    )
    def sc_agg(h_hbm, sd_hbm, zeros_hbm, out_hbm,
               idx0_v, idx1_v, rows0_v, rows1_v, agg_sh,
               gsem0, gsem1, isem0, isem1):
        cid = lax.axis_index("c")
        sid = lax.axis_index("s")
        w = cid * 16 + sid
        pltpu.sync_copy(zeros_hbm, agg_sh.at[pl.ds(sid * ZR, ZR)])
        plsc.subcore_barrier()

        idx = (idx0_v, idx1_v)
        rows = (rows0_v, rows1_v)
        gsem = (gsem0, gsem1)
        isem = (isem0, isem1)

        pltpu.sync_copy(sd_hbm.at[w, 0], idx[0])
        pltpu.async_copy(sd_hbm.at[w, 1], idx[1], isem[1])
        pltpu.async_copy(h_hbm.at[idx[0].at[0]], rows[0], gsem[0])

        def body(p, carry):
            for b in (0, 1):
                g = 2 * p + b
                nb = 1 - b
                pltpu.make_async_copy(h_hbm.at[idx[b].at[0]], rows[b],
                                      gsem[b]).wait()

                @pl.when(g + 1 < ch_per_w)
                def _():
                    pltpu.make_async_copy(sd_hbm.at[w, g + 1], idx[nb],
                                          isem[nb]).wait()
                    pltpu.async_copy(h_hbm.at[idx[nb].at[0]], rows[nb],
                                     gsem[nb])

                pltpu.sync_copy(rows[b], agg_sh.at[idx[b].at[1]], add=True)

                @pl.when(g + 2 < ch_per_w)
                def _():
                    pltpu.async_copy(sd_hbm.at[w, g + 2], idx[b], isem[b])
            return carry

        lax.fori_loop(0, ch_per_w // 2, body, 0)
        plsc.subcore_barrier()
        pltpu.sync_copy(agg_sh.at[pl.ds(sid * ZR, ZR)],
                        out_hbm.at[cid, pl.ds(sid * ZR, ZR)])

    return sc_agg


def _mlp_body(h_ref, a_ref, m_ref, p_ref, vc_ref, w1_ref, b1_ref,
              w2_ref, b2_ref, o_ref, vo_ref, *, last):
    i = pl.program_id(0)
    z = h_ref[...] + a_ref[0] + a_ref[1]
    z = z + jnp.dot(m_ref[...], vc_ref[...], preferred_element_type=jnp.float32)
    t = jnp.dot(z, w1_ref[...], preferred_element_type=jnp.float32) + b1_ref[...]
    t = jnp.maximum(t, 0.0)
    y = jnp.dot(t, w2_ref[...], preferred_element_type=jnp.float32) + b2_ref[...]
    if not last:
        y = jnp.maximum(y, 0.0)
    o_ref[...] = y
    part = lax.dot_general(p_ref[...], y, (((0,), (0,)), ((), ())),
                           preferred_element_type=jnp.float32)

    @pl.when(i == 0)
    def _():
        vo_ref[...] = jnp.zeros_like(vo_ref)

    vo_ref[...] += part


def _mlp(h, agg, m, p, vcat, w1t, b1, w2t, b2, last):
    grid = (RP // BR,)
    return pl.pallas_call(
        functools.partial(_mlp_body, last=last),
        grid=grid,
        in_specs=[
            pl.BlockSpec((BR, NDIM), lambda i: (i, 0)),
            pl.BlockSpec((2, BR, NDIM), lambda i: (0, i, 0)),
            pl.BlockSpec((BR, 2 * NGRAPH), lambda i: (i, 0)),
            pl.BlockSpec((BR, 2 * NGRAPH), lambda i: (i, 0)),
            pl.BlockSpec((2 * NGRAPH, NDIM), lambda i: (0, 0)),
            pl.BlockSpec((NDIM, NDIM), lambda i: (0, 0)),
            pl.BlockSpec((1, NDIM), lambda i: (0, 0)),
            pl.BlockSpec((NDIM, NDIM), lambda i: (0, 0)),
            pl.BlockSpec((1, NDIM), lambda i: (0, 0)),
        ],
        out_specs=[
            pl.BlockSpec((BR, NDIM), lambda i: (i, 0)),
            pl.BlockSpec((2 * NGRAPH, NDIM), lambda i: (0, 0)),
        ],
        out_shape=[
            jax.ShapeDtypeStruct((RP, NDIM), jnp.float32),
            jax.ShapeDtypeStruct((2 * NGRAPH, NDIM), jnp.float32),
        ],
    )(h, agg, m, p, vcat, w1t, b1, w2t, b2)


def _vcat0_body(h_ref, p_ref, vo_ref):
    i = pl.program_id(0)

    @pl.when(i == 0)
    def _():
        vo_ref[...] = jnp.zeros_like(vo_ref)

    vo_ref[...] += lax.dot_general(p_ref[...], h_ref[...],
                                   (((0,), (0,)), ((), ())),
                                   preferred_element_type=jnp.float32)


def _vcat0(h, p):
    return pl.pallas_call(
        _vcat0_body,
        grid=(RP // BR,),
        in_specs=[
            pl.BlockSpec((BR, NDIM), lambda i: (i, 0)),
            pl.BlockSpec((BR, 2 * NGRAPH), lambda i: (i, 0)),
        ],
        out_specs=pl.BlockSpec((2 * NGRAPH, NDIM), lambda i: (0, 0)),
        out_shape=jax.ShapeDtypeStruct((2 * NGRAPH, NDIM), jnp.float32),
    )(h, p)


def _final_body(hv_ref, wl_ref, bl_ref, o_ref):
    z = jnp.maximum(hv_ref[...], 0.0)
    o_ref[...] = (jnp.dot(z, wl_ref[...], preferred_element_type=jnp.float32)
                  + bl_ref[...])


def _final(hv, wlt, bl):
    return pl.pallas_call(
        _final_body,
        out_shape=jax.ShapeDtypeStruct((NGRAPH, NDIM), jnp.float32),
    )(hv, wlt, bl)


def kernel(x, edge_index, batch, W1, b1, W2, b2, Wl, bl):
    n, d = x.shape
    e = edge_index.shape[1]
    idt = jnp.int32
    src = edge_index[0].astype(idt)
    dst = edge_index[1].astype(idt)
    ch_per_w = -(-e // (NWORK * CHUNK))
    ch_per_w += ch_per_w % 2
    pad = NWORK * CHUNK * ch_per_w - e
    pk = jnp.arange(pad, dtype=idt)
    src = jnp.concatenate([src, (pk * 131) % jnp.int32(n)])
    dst = jnp.concatenate([dst, NTOT + pk % jnp.int32(RP - NTOT)])
    sd = jnp.stack([src.reshape(NWORK, ch_per_w, CHUNK),
                    dst.reshape(NWORK, ch_per_w, CHUNK)], axis=2)

    h = jnp.zeros((RP, NDIM), jnp.float32).at[:n].set(x)
    zeros = jnp.zeros((ZR, NDIM), jnp.float32)
    gids = jnp.arange(NGRAPH, dtype=idt)
    bpad = jnp.concatenate([batch.astype(idt), jnp.full((RP - n,), -1, idt)])
    m_real = (bpad[:, None] == gids[None, :]).astype(jnp.float32)
    rows_i = jnp.arange(RP, dtype=idt)
    m_virt = ((rows_i[:, None] - n) == gids[None, :]).astype(jnp.float32)
    m = jnp.concatenate([m_real, m_virt], axis=1)
    p = jnp.concatenate([m_virt, m_real], axis=1)

    w1t = jnp.swapaxes(W1, 1, 2)
    w2t = jnp.swapaxes(W2, 1, 2)
    nl = W1.shape[0]
    b1r = b1.reshape(nl, 1, NDIM)
    b2r = b2.reshape(nl, 1, NDIM)

    sc_agg = _make_sc_agg(ch_per_w)
    vcat = _vcat0(h, p)
    for l in range(nl):
        agg = sc_agg(h, sd, zeros)
        h, vcat = _mlp(h, agg, m, p, vcat, w1t[l], b1r[l], w2t[l], b2r[l],
                       last=(l == nl - 1))

    hv = lax.slice(vcat, (0, 0), (NGRAPH, NDIM))
    return _final(hv, Wl.T, bl.reshape(1, NDIM))
